# baseline (speedup 1.0000x reference)
import functools

import jax
import jax.numpy as jnp
from jax import lax
from jax.experimental import pallas as pl
from jax.experimental.pallas import tpu as pltpu
from jax.experimental.pallas import tpu_sc as plsc

B = 16384
F = 26
V = 100000
D = 16
ND = 13
EPS = 1e-3

NC = 2
NS = 16
NW = NC * NS
RPW = B // NW
C = 128
CHUNKS = RPW // C
VL = 16
WPW = RPW * F
WPC = C * F
GPC = WPC // 128
PAT = 208


def _bi_interact_sc(flat_table, idx_flat, offs_pat):
    mesh = plsc.VectorSubcoreMesh(core_axis_name="c", subcore_axis_name="s")

    @functools.partial(
        pl.kernel,
        mesh=mesh,
        compiler_params=pltpu.CompilerParams(use_tc_tiling_on_sc=False),
        out_type=jax.ShapeDtypeStruct((B, D), jnp.float32),
        scratch_types=[
            pltpu.VMEM((WPW,), jnp.int32),
            pltpu.VMEM((PAT,), jnp.int32),
            pltpu.VMEM((WPC, D), jnp.float32),
            pltpu.VMEM((C, D), jnp.float32),
            pltpu.SemaphoreType.DMA,
        ],
    )
    def sc_kernel(table_hbm, idx_hbm, pat_hbm, out_hbm,
                  idx_v, pat_v, rows_v, out_v, sem):
        w = lax.axis_index("s") * NC + lax.axis_index("c")
        pltpu.sync_copy(pat_hbm, pat_v)
        pltpu.sync_copy(idx_hbm.at[pl.ds(w * WPW, WPW)], idx_v)

        @pl.loop(0, WPW // PAT)
        def _ofs(j):
            for k in range(PAT // VL):
                sl = pl.ds(j * PAT + k * VL, VL)
                idx_v[sl] = idx_v[sl] + pat_v[pl.ds(k * VL, VL)]

        @pl.loop(0, CHUNKS)
        def _chunk(kk):
            cps = [
                pltpu.async_copy(
                    table_hbm.at[idx_v.at[pl.ds(kk * WPC + g * 128, 128)]],
                    rows_v.at[pl.ds(g * 128, 128), :],
                    sem,
                )
                for g in range(GPC)
            ]
            for cp in cps:
                cp.wait()

            @pl.loop(0, C)
            def _row(r):
                base = r * F
                v = rows_v[base, :]
                acc = v
                asq = v * v
                for f in range(1, F):
                    v = rows_v[base + f, :]
                    acc = acc + v
                    asq = asq + v * v
                out_v[r, :] = (acc * acc - asq) * 0.5

            pltpu.sync_copy(out_v, out_hbm.at[pl.ds(w * RPW + kk * C, C), :])

    return sc_kernel(flat_table, idx_flat, offs_pat)


RS = 2048
NBS = B // RS


def _stats_body(d_ref, c_ref, gd_ref, bd_ref, gc_ref, bc_ref,
                ad_ref, bd2_ref, ac_ref, bc2_ref):
    def accum(i, carry):
        sd, qd, sc_, qc = carry
        dch = d_ref[pl.ds(i * RS, RS), :]
        cch = c_ref[pl.ds(i * RS, RS), :]
        return (
            sd + jnp.sum(dch, axis=0, keepdims=True),
            qd + jnp.sum(dch * dch, axis=0, keepdims=True),
            sc_ + jnp.sum(cch, axis=0, keepdims=True),
            qc + jnp.sum(cch * cch, axis=0, keepdims=True),
        )

    z_d = jnp.zeros((1, ND), jnp.float32)
    z_c = jnp.zeros((1, D), jnp.float32)
    sd, qd, sc_, qc = lax.fori_loop(0, NBS, accum, (z_d, z_d, z_c, z_c))
    md = sd / B
    vd = qd / B - md * md
    ad = gd_ref[...] * lax.rsqrt(vd + EPS)
    bd2 = bd_ref[...] - md * ad
    mc = sc_ / B
    vc = qc / B - mc * mc
    ac = gc_ref[...] * lax.rsqrt(vc + EPS)
    bc2 = bc_ref[...] - mc * ac
    ad_ref[...] = ad
    bd2_ref[...] = bd2
    ac_ref[...] = ac
    bc2_ref[...] = bc2


def _stats_tc(dense, cross, gd, bd, gc, bc):
    out_types = (
        jax.ShapeDtypeStruct((1, ND), jnp.float32),
        jax.ShapeDtypeStruct((1, ND), jnp.float32),
        jax.ShapeDtypeStruct((1, D), jnp.float32),
        jax.ShapeDtypeStruct((1, D), jnp.float32),
    )
    return pl.pallas_call(
        _stats_body,
        out_shape=out_types,
    )(dense, cross, gd, bd, gc, bc)


RM = 1024
NBM = B // RM


def _mlp_body(d_ref, c_ref, ad_ref, bd_ref, ac_ref, bc_ref,
              w1d_ref, w1c_ref, b1_ref, w2_ref, b2_ref, w3_ref, b3_ref,
              w4_ref, b4_ref, o_ref):
    hp = lax.Precision.HIGHEST
    xd = d_ref[...] * ad_ref[...] + bd_ref[...]
    xc = c_ref[...] * ac_ref[...] + bc_ref[...]
    h = jnp.dot(xd, w1d_ref[...], precision=hp) \
        + jnp.dot(xc, w1c_ref[...], precision=hp) + b1_ref[...]
    h = jnp.maximum(h, 0.0)
    h = jnp.maximum(jnp.dot(h, w2_ref[...], precision=hp) + b2_ref[...], 0.0)
    h = jnp.maximum(jnp.dot(h, w3_ref[...], precision=hp) + b3_ref[...], 0.0)
    o_ref[...] = jax.nn.sigmoid(jnp.dot(h, w4_ref[...], precision=hp)
                                + b4_ref[...])


def _mlp_tc(dense, cross, ad, bd2, ac, bc2, w1d, w1c, b1, w2, b2, w3, b3,
            w4, b4):
    full = lambda shape: pl.BlockSpec(shape, lambda i: (0, 0))
    return pl.pallas_call(
        _mlp_body,
        grid=(NBM,),
        in_specs=[
            pl.BlockSpec((RM, ND), lambda i: (i, 0)),
            pl.BlockSpec((RM, D), lambda i: (i, 0)),
            full((1, ND)), full((1, ND)), full((1, D)), full((1, D)),
            full((ND, 256)), full((D, 256)), full((1, 256)),
            full((256, 128)), full((1, 128)),
            full((128, 64)), full((1, 64)),
            full((64, 1)), full((1, 1)),
        ],
        out_specs=pl.BlockSpec((RM, 1), lambda i: (i, 0)),
        out_shape=jax.ShapeDtypeStruct((B, 1), jnp.float32),
    )(dense, cross, ad, bd2, ac, bc2, w1d, w1c, b1, w2, b2, w3, b3, w4, b4)


def kernel(dense_inputs, sparse_inputs, emb, gamma, beta,
           W1, b1, W2, b2, W3, b3, W4, b4):
    flat_table = emb.reshape(F * V, D)
    idx_flat = sparse_inputs.reshape(B * F)
    offs_pat = jnp.tile(jnp.arange(F, dtype=jnp.int32) * V, PAT // F)

    cross = _bi_interact_sc(flat_table, idx_flat, offs_pat)

    gd = gamma[:ND].reshape(1, ND)
    gc = gamma[ND:].reshape(1, D)
    bd = beta[:ND].reshape(1, ND)
    bc = beta[ND:].reshape(1, D)
    ad, bd2, ac, bc2 = _stats_tc(dense_inputs, cross, gd, bd, gc, bc)

    return _mlp_tc(
        dense_inputs, cross, ad, bd2, ac, bc2,
        W1[:ND], W1[ND:], b1.reshape(1, 256),
        W2, b2.reshape(1, 128), W3, b3.reshape(1, 64),
        W4, b4.reshape(1, 1),
    )


# speedup vs baseline: 1.0153x; 1.0153x over previous
import functools

import jax
import jax.numpy as jnp
from jax import lax
from jax.experimental import pallas as pl
from jax.experimental.pallas import tpu as pltpu
from jax.experimental.pallas import tpu_sc as plsc

B = 16384
F = 26
V = 100000
D = 16
ND = 13
EPS = 1e-3

NC = 2
NS = 16
NW = NC * NS
RPW = B // NW
C = 128
CHUNKS = RPW // C
VL = 16
def _bi_interact_sc(table, idxT):
    mesh = plsc.VectorSubcoreMesh(core_axis_name="c", subcore_axis_name="s")

    @functools.partial(
        pl.kernel,
        mesh=mesh,
        compiler_params=pltpu.CompilerParams(use_tc_tiling_on_sc=False),
        out_type=jax.ShapeDtypeStruct((B, D), jnp.float32),
        scratch_types=[
            pltpu.VMEM((F, RPW), jnp.int32),
            pltpu.VMEM((F * C, D), jnp.float32),
            pltpu.VMEM((C, D), jnp.float32),
            pltpu.SemaphoreType.DMA,
        ],
    )
    def sc_kernel(table_hbm, idx_hbm, out_hbm, idx_v, rows_v, out_v, sem):
        w = lax.axis_index("s") * NC + lax.axis_index("c")
        pltpu.sync_copy(idx_hbm.at[:, pl.ds(w * RPW, RPW)], idx_v)

        @pl.loop(0, CHUNKS)
        def _chunk(kk):
            cps = [
                pltpu.async_copy(
                    table_hbm.at[f].at[idx_v.at[f, pl.ds(kk * C, C)]],
                    rows_v.at[pl.ds(f * C, C), :],
                    sem,
                )
                for f in range(F)
            ]
            for cp in cps:
                cp.wait()

            @pl.loop(0, C)
            def _row(r):
                v = rows_v[r, :]
                acc = v
                asq = v * v
                for f in range(1, F):
                    v = rows_v[f * C + r, :]
                    acc = acc + v
                    asq = asq + v * v
                out_v[r, :] = (acc * acc - asq) * 0.5

            pltpu.sync_copy(out_v, out_hbm.at[pl.ds(w * RPW + kk * C, C), :])

    return sc_kernel(table, idxT)


RS = 2048
NBS = B // RS


def _stats_body(d_ref, c_ref, gd_ref, bd_ref, gc_ref, bc_ref,
                ad_ref, bd2_ref, ac_ref, bc2_ref):
    def accum(i, carry):
        sd, qd, sc_, qc = carry
        dch = d_ref[pl.ds(i * RS, RS), :]
        cch = c_ref[pl.ds(i * RS, RS), :]
        return (
            sd + jnp.sum(dch, axis=0, keepdims=True),
            qd + jnp.sum(dch * dch, axis=0, keepdims=True),
            sc_ + jnp.sum(cch, axis=0, keepdims=True),
            qc + jnp.sum(cch * cch, axis=0, keepdims=True),
        )

    z_d = jnp.zeros((1, ND), jnp.float32)
    z_c = jnp.zeros((1, D), jnp.float32)
    sd, qd, sc_, qc = lax.fori_loop(0, NBS, accum, (z_d, z_d, z_c, z_c))
    md = sd / B
    vd = qd / B - md * md
    ad = gd_ref[...] * lax.rsqrt(vd + EPS)
    bd2 = bd_ref[...] - md * ad
    mc = sc_ / B
    vc = qc / B - mc * mc
    ac = gc_ref[...] * lax.rsqrt(vc + EPS)
    bc2 = bc_ref[...] - mc * ac
    ad_ref[...] = ad
    bd2_ref[...] = bd2
    ac_ref[...] = ac
    bc2_ref[...] = bc2


def _stats_tc(dense, cross, gd, bd, gc, bc):
    out_types = (
        jax.ShapeDtypeStruct((1, ND), jnp.float32),
        jax.ShapeDtypeStruct((1, ND), jnp.float32),
        jax.ShapeDtypeStruct((1, D), jnp.float32),
        jax.ShapeDtypeStruct((1, D), jnp.float32),
    )
    return pl.pallas_call(
        _stats_body,
        out_shape=out_types,
    )(dense, cross, gd, bd, gc, bc)


RM = 1024
NBM = B // RM


def _mlp_body(d_ref, c_ref, ad_ref, bd_ref, ac_ref, bc_ref,
              w1d_ref, w1c_ref, b1_ref, w2_ref, b2_ref, w3_ref, b3_ref,
              w4_ref, b4_ref, o_ref):
    hp = lax.Precision.HIGHEST
    xd = d_ref[...] * ad_ref[...] + bd_ref[...]
    xc = c_ref[...] * ac_ref[...] + bc_ref[...]
    h = jnp.dot(xd, w1d_ref[...], precision=hp) \
        + jnp.dot(xc, w1c_ref[...], precision=hp) + b1_ref[...]
    h = jnp.maximum(h, 0.0)
    h = jnp.maximum(jnp.dot(h, w2_ref[...], precision=hp) + b2_ref[...], 0.0)
    h = jnp.maximum(jnp.dot(h, w3_ref[...], precision=hp) + b3_ref[...], 0.0)
    o_ref[...] = jax.nn.sigmoid(jnp.dot(h, w4_ref[...], precision=hp)
                                + b4_ref[...])


def _mlp_tc(dense, cross, ad, bd2, ac, bc2, w1d, w1c, b1, w2, b2, w3, b3,
            w4, b4):
    full = lambda shape: pl.BlockSpec(shape, lambda i: (0, 0))
    return pl.pallas_call(
        _mlp_body,
        grid=(NBM,),
        in_specs=[
            pl.BlockSpec((RM, ND), lambda i: (i, 0)),
            pl.BlockSpec((RM, D), lambda i: (i, 0)),
            full((1, ND)), full((1, ND)), full((1, D)), full((1, D)),
            full((ND, 256)), full((D, 256)), full((1, 256)),
            full((256, 128)), full((1, 128)),
            full((128, 64)), full((1, 64)),
            full((64, 1)), full((1, 1)),
        ],
        out_specs=pl.BlockSpec((RM, 1), lambda i: (i, 0)),
        out_shape=jax.ShapeDtypeStruct((B, 1), jnp.float32),
    )(dense, cross, ad, bd2, ac, bc2, w1d, w1c, b1, w2, b2, w3, b3, w4, b4)


def kernel(dense_inputs, sparse_inputs, emb, gamma, beta,
           W1, b1, W2, b2, W3, b3, W4, b4):
    idxT = jnp.swapaxes(sparse_inputs, 0, 1)
    cross = _bi_interact_sc(emb, idxT)

    gd = gamma[:ND].reshape(1, ND)
    gc = gamma[ND:].reshape(1, D)
    bd = beta[:ND].reshape(1, ND)
    bc = beta[ND:].reshape(1, D)
    ad, bd2, ac, bc2 = _stats_tc(dense_inputs, cross, gd, bd, gc, bc)

    return _mlp_tc(
        dense_inputs, cross, ad, bd2, ac, bc2,
        W1[:ND], W1[ND:], b1.reshape(1, 256),
        W2, b2.reshape(1, 128), W3, b3.reshape(1, 64),
        W4, b4.reshape(1, 1),
    )
